# baseline (device time: 23474 ns/iter reference)
import jax
import jax.numpy as jnp
from jax import lax
from jax.experimental import pallas as pl
from jax.experimental.pallas import tpu as pltpu

NCHUNK = 16


def kernel(x):
    m, n = x.shape
    n_half = n // 2
    ch = m // NCHUNK
    nk = NCHUNK // 2

    def body(x_ref, out_ref, xsend_sems, xrecv_sems, ysend_sems, yrecv_sems,
             copy_sem):
        mx = lax.axis_index("x")
        my = lax.axis_index("y")
        ox = 1 - mx
        oy = 1 - my

        local_copy = pltpu.make_async_copy(
            x_ref.at[:, pl.ds(mx * n_half, n_half)],
            out_ref.at[pl.ds(mx * m, m), :],
            copy_sem,
        )
        local_copy.start()

        barrier_sem = pltpu.get_barrier_semaphore()
        pl.semaphore_signal(
            barrier_sem, inc=1,
            device_id=(ox, my), device_id_type=pl.DeviceIdType.MESH,
        )
        pl.semaphore_signal(
            barrier_sem, inc=1,
            device_id=(mx, oy), device_id_type=pl.DeviceIdType.MESH,
        )
        pl.semaphore_wait(barrier_sem, 2)

        row0 = ox * m

        xsends = []
        for k in range(nk):
            c = 2 * k + my
            rdma = pltpu.make_async_remote_copy(
                src_ref=x_ref.at[pl.ds(c * ch, ch), pl.ds(ox * n_half, n_half)],
                dst_ref=out_ref.at[pl.ds(mx * m + c * ch, ch), :],
                send_sem=xsend_sems.at[k],
                recv_sem=xrecv_sems.at[k],
                device_id=(ox, my),
                device_id_type=pl.DeviceIdType.MESH,
            )
            rdma.start()
            xsends.append(rdma)

        fwds = []
        for k in range(nk):
            c = 2 * k + my
            xsends[k].wait_recv()
            sl = out_ref.at[pl.ds(row0 + c * ch, ch), :]
            fwd = pltpu.make_async_remote_copy(
                src_ref=sl,
                dst_ref=sl,
                send_sem=ysend_sems.at[k],
                recv_sem=yrecv_sems.at[k],
                device_id=(mx, oy),
                device_id_type=pl.DeviceIdType.MESH,
            )
            fwd.start()
            fwds.append(fwd)

        for k in range(nk):
            fwds[k].wait_recv()
        for k in range(nk):
            xsends[k].wait_send()
            fwds[k].wait_send()
        local_copy.wait()

    return pl.pallas_call(
        body,
        out_shape=jax.ShapeDtypeStruct((2 * m, n_half), x.dtype),
        in_specs=[pl.BlockSpec(memory_space=pl.ANY)],
        out_specs=pl.BlockSpec(memory_space=pl.ANY),
        scratch_shapes=[
            pltpu.SemaphoreType.DMA((nk,)),
            pltpu.SemaphoreType.DMA((nk,)),
            pltpu.SemaphoreType.DMA((nk,)),
            pltpu.SemaphoreType.DMA((nk,)),
            pltpu.SemaphoreType.DMA,
        ],
        compiler_params=pltpu.CompilerParams(collective_id=0),
    )(x)


# device time: 23246 ns/iter; 1.0098x vs baseline; 1.0098x over previous
import jax
import jax.numpy as jnp
from jax import lax
from jax.experimental import pallas as pl
from jax.experimental.pallas import tpu as pltpu

NCHUNK = 16


def kernel(x):
    m, n = x.shape
    n_half = n // 2
    ch = m // NCHUNK
    nk = NCHUNK // 2

    def body(x_ref, out_ref, xsend_sems, xrecv_sems, ysend_sems, yrecv_sems,
             copy_sem):
        mx = lax.axis_index("x")
        my = lax.axis_index("y")
        ox = 1 - mx
        oy = 1 - my

        local_copy = pltpu.make_async_copy(
            x_ref.at[:, pl.ds(mx * n_half, n_half)],
            out_ref.at[pl.ds(mx * m, m), :],
            copy_sem,
        )
        local_copy.start()

        barrier_sem = pltpu.get_barrier_semaphore()
        pl.semaphore_signal(
            barrier_sem, inc=1,
            device_id=(ox, my), device_id_type=pl.DeviceIdType.MESH,
        )
        pl.semaphore_signal(
            barrier_sem, inc=1,
            device_id=(mx, oy), device_id_type=pl.DeviceIdType.MESH,
        )
        pl.semaphore_wait(barrier_sem, 2)

        row0 = ox * m

        xsends = []
        for k in range(nk):
            c = 2 * k + my
            rdma = pltpu.make_async_remote_copy(
                src_ref=x_ref.at[pl.ds(c * ch, ch), pl.ds(ox * n_half, n_half)],
                dst_ref=out_ref.at[pl.ds(mx * m + c * ch, ch), :],
                send_sem=xsend_sems.at[k],
                recv_sem=xrecv_sems.at[k],
                device_id=(ox, my),
                device_id_type=pl.DeviceIdType.MESH,
            )
            rdma.start()
            xsends.append(rdma)

        fwds = []
        for k in range(nk):
            c = 2 * k + my
            xsends[k].wait_recv()
            sl = out_ref.at[pl.ds(row0 + c * ch, ch), :]
            fwd = pltpu.make_async_remote_copy(
                src_ref=sl,
                dst_ref=sl,
                send_sem=ysend_sems.at[k],
                recv_sem=yrecv_sems.at[k],
                device_id=(mx, oy),
                device_id_type=pl.DeviceIdType.MESH,
            )
            fwd.start()
            fwds.append(fwd)

        for k in range(nk):
            fwds[k].wait_recv()
        for k in range(nk):
            xsends[k].wait_send()
            fwds[k].wait_send()
        local_copy.wait()

    return pl.pallas_call(
        body,
        out_shape=jax.ShapeDtypeStruct((2 * m, n_half), x.dtype),
        in_specs=[pl.BlockSpec(memory_space=pltpu.MemorySpace.HBM)],
        out_specs=pl.BlockSpec(memory_space=pltpu.MemorySpace.HBM),
        scratch_shapes=[
            pltpu.SemaphoreType.DMA((nk,)),
            pltpu.SemaphoreType.DMA((nk,)),
            pltpu.SemaphoreType.DMA((nk,)),
            pltpu.SemaphoreType.DMA((nk,)),
            pltpu.SemaphoreType.DMA,
        ],
        compiler_params=pltpu.CompilerParams(collective_id=0),
    )(x)


# device time: 21353 ns/iter; 1.0993x vs baseline; 1.0887x over previous
import jax
import jax.numpy as jnp
from jax import lax
from jax.experimental import pallas as pl
from jax.experimental.pallas import tpu as pltpu

NCHUNK = 16


def kernel(x):
    m, n = x.shape
    n_half = n // 2
    ch = m // NCHUNK
    nk = NCHUNK // 2

    def body(x_ref, out_ref, xsend_sems, xrecv_sems, ysend_sems, yrecv_sems,
             copy_sem):
        mx = lax.axis_index("x")
        my = lax.axis_index("y")
        ox = 1 - mx
        oy = 1 - my

        local_copy = pltpu.make_async_copy(
            x_ref.at[:, pl.ds(mx * n_half, n_half)],
            out_ref.at[pl.ds(mx * m, m), :],
            copy_sem,
        )
        local_copy.start()

        barrier_sem = pltpu.get_barrier_semaphore()
        pl.semaphore_signal(
            barrier_sem, inc=1,
            device_id=(ox, my), device_id_type=pl.DeviceIdType.MESH,
        )
        pl.semaphore_signal(
            barrier_sem, inc=1,
            device_id=(mx, oy), device_id_type=pl.DeviceIdType.MESH,
        )
        pl.semaphore_wait(barrier_sem, 2)

        row0 = ox * m

        xsends = []
        for k in range(nk):
            c = 2 * k + my
            rdma = pltpu.make_async_remote_copy(
                src_ref=x_ref.at[pl.ds(c * ch, ch), pl.ds(ox * n_half, n_half)],
                dst_ref=out_ref.at[pl.ds(mx * m + c * ch, ch), :],
                send_sem=xsend_sems.at[k],
                recv_sem=xrecv_sems.at[k],
                device_id=(ox, my),
                device_id_type=pl.DeviceIdType.MESH,
            )
            rdma.start()
            xsends.append(rdma)

        fwds = []
        for k in range(nk):
            c = 2 * k + my
            xsends[k].wait_recv()
            sl = out_ref.at[pl.ds(row0 + c * ch, ch), :]
            fwd = pltpu.make_async_remote_copy(
                src_ref=sl,
                dst_ref=sl,
                send_sem=ysend_sems.at[k],
                recv_sem=yrecv_sems.at[k],
                device_id=(mx, oy),
                device_id_type=pl.DeviceIdType.MESH,
            )
            fwd.start()
            fwds.append(fwd)

        for k in range(nk):
            fwds[k].wait_recv()
        for k in range(nk):
            xsends[k].wait_send()
            fwds[k].wait_send()
        local_copy.wait()

    x = pltpu.with_memory_space_constraint(x, pltpu.MemorySpace.HBM)
    return pl.pallas_call(
        body,
        out_shape=jax.ShapeDtypeStruct((2 * m, n_half), x.dtype),
        in_specs=[pl.BlockSpec(memory_space=pltpu.MemorySpace.HBM)],
        out_specs=pl.BlockSpec(memory_space=pltpu.MemorySpace.VMEM),
        scratch_shapes=[
            pltpu.SemaphoreType.DMA((nk,)),
            pltpu.SemaphoreType.DMA((nk,)),
            pltpu.SemaphoreType.DMA((nk,)),
            pltpu.SemaphoreType.DMA((nk,)),
            pltpu.SemaphoreType.DMA,
        ],
        compiler_params=pltpu.CompilerParams(collective_id=0),
    )(x)


# device time: 20227 ns/iter; 1.1605x vs baseline; 1.0557x over previous
import jax
import jax.numpy as jnp
from jax import lax
from jax.experimental import pallas as pl
from jax.experimental.pallas import tpu as pltpu

NCHUNK = 32
NF = 14
ND = NCHUNK - 2 * NF


def kernel(x):
    m, n = x.shape
    n_half = n // 2
    ch = m // NCHUNK
    nx = NF + ND

    def body(x_ref, out_ref, xsend_sems, xrecv_sems, ysend_sems, yrecv_sems,
             copy_sem):
        mx = lax.axis_index("x")
        my = lax.axis_index("y")
        ox = 1 - mx
        oy = 1 - my

        local_copy = pltpu.make_async_copy(
            x_ref.at[:, pl.ds(mx * n_half, n_half)],
            out_ref.at[pl.ds(mx * m, m), :],
            copy_sem,
        )
        local_copy.start()

        barrier_sem = pltpu.get_barrier_semaphore()
        pl.semaphore_signal(
            barrier_sem, inc=1,
            device_id=(ox, my), device_id_type=pl.DeviceIdType.MESH,
        )
        pl.semaphore_signal(
            barrier_sem, inc=1,
            device_id=(mx, oy), device_id_type=pl.DeviceIdType.MESH,
        )
        pl.semaphore_wait(barrier_sem, 2)

        row0 = ox * m

        def chunk_idx(k):
            return jnp.where(k < NF, NF * my + k, 2 * NF + (k - NF))

        xsends = []
        for k in range(nx):
            i = chunk_idx(k)
            rdma = pltpu.make_async_remote_copy(
                src_ref=x_ref.at[pl.ds(i * ch, ch), pl.ds(ox * n_half, n_half)],
                dst_ref=out_ref.at[pl.ds(mx * m + i * ch, ch), :],
                send_sem=xsend_sems.at[k],
                recv_sem=xrecv_sems.at[k],
                device_id=(ox, my),
                device_id_type=pl.DeviceIdType.MESH,
            )
            rdma.start()
            xsends.append(rdma)

        fwds = []
        for k in range(NF):
            i = NF * my + k
            xsends[k].wait_recv()
            sl = out_ref.at[pl.ds(row0 + i * ch, ch), :]
            fwd = pltpu.make_async_remote_copy(
                src_ref=sl,
                dst_ref=sl,
                send_sem=ysend_sems.at[k],
                recv_sem=yrecv_sems.at[k],
                device_id=(mx, oy),
                device_id_type=pl.DeviceIdType.MESH,
            )
            fwd.start()
            fwds.append(fwd)

        for k in range(NF, nx):
            xsends[k].wait_recv()
        for k in range(NF):
            fwds[k].wait_recv()
        for k in range(nx):
            xsends[k].wait_send()
        for k in range(NF):
            fwds[k].wait_send()
        local_copy.wait()

    x = pltpu.with_memory_space_constraint(x, pltpu.MemorySpace.HBM)
    return pl.pallas_call(
        body,
        out_shape=jax.ShapeDtypeStruct((2 * m, n_half), x.dtype),
        in_specs=[pl.BlockSpec(memory_space=pltpu.MemorySpace.HBM)],
        out_specs=pl.BlockSpec(memory_space=pltpu.MemorySpace.VMEM),
        scratch_shapes=[
            pltpu.SemaphoreType.DMA((NF + ND,)),
            pltpu.SemaphoreType.DMA((NF + ND,)),
            pltpu.SemaphoreType.DMA((NF,)),
            pltpu.SemaphoreType.DMA((NF,)),
            pltpu.SemaphoreType.DMA,
        ],
        compiler_params=pltpu.CompilerParams(collective_id=0),
    )(x)


# device time: 20049 ns/iter; 1.1708x vs baseline; 1.0089x over previous
import jax
import jax.numpy as jnp
from jax import lax
from jax.experimental import pallas as pl
from jax.experimental.pallas import tpu as pltpu

NCHUNK = 32
NF = 14
ND = NCHUNK - 2 * NF


def kernel(x):
    m, n = x.shape
    n_half = n // 2
    ch = m // NCHUNK
    nx = NF + ND

    n_stage = 2

    def body(x_ref, out_ref, stage_ref, xsend_sems, xrecv_sems, ysend_sems,
             yrecv_sems, copy_sem, stage_sems):
        mx = lax.axis_index("x")
        my = lax.axis_index("y")
        ox = 1 - mx
        oy = 1 - my

        stages = []
        for j in range(n_stage):
            i = NF * my + j
            cp = pltpu.make_async_copy(
                x_ref.at[pl.ds(i * ch, ch), pl.ds(ox * n_half, n_half)],
                stage_ref.at[j],
                stage_sems.at[j],
            )
            cp.start()
            stages.append(cp)

        local_copy = pltpu.make_async_copy(
            x_ref.at[:, pl.ds(mx * n_half, n_half)],
            out_ref.at[pl.ds(mx * m, m), :],
            copy_sem,
        )
        local_copy.start()

        barrier_sem = pltpu.get_barrier_semaphore()
        pl.semaphore_signal(
            barrier_sem, inc=1,
            device_id=(ox, my), device_id_type=pl.DeviceIdType.MESH,
        )
        pl.semaphore_signal(
            barrier_sem, inc=1,
            device_id=(mx, oy), device_id_type=pl.DeviceIdType.MESH,
        )
        pl.semaphore_wait(barrier_sem, 2)

        row0 = ox * m

        def chunk_idx(k):
            return jnp.where(k < NF, NF * my + k, 2 * NF + (k - NF))

        xsends = []
        for k in range(nx):
            i = chunk_idx(k)
            if k < n_stage:
                stages[k].wait()
                src = stage_ref.at[k]
            else:
                src = x_ref.at[pl.ds(i * ch, ch), pl.ds(ox * n_half, n_half)]
            rdma = pltpu.make_async_remote_copy(
                src_ref=src,
                dst_ref=out_ref.at[pl.ds(mx * m + i * ch, ch), :],
                send_sem=xsend_sems.at[k],
                recv_sem=xrecv_sems.at[k],
                device_id=(ox, my),
                device_id_type=pl.DeviceIdType.MESH,
            )
            rdma.start()
            xsends.append(rdma)

        fwds = []
        for k in range(NF):
            i = NF * my + k
            xsends[k].wait_recv()
            sl = out_ref.at[pl.ds(row0 + i * ch, ch), :]
            fwd = pltpu.make_async_remote_copy(
                src_ref=sl,
                dst_ref=sl,
                send_sem=ysend_sems.at[k],
                recv_sem=yrecv_sems.at[k],
                device_id=(mx, oy),
                device_id_type=pl.DeviceIdType.MESH,
            )
            fwd.start()
            fwds.append(fwd)

        for k in range(NF, nx):
            xsends[k].wait_recv()
        for k in range(NF):
            fwds[k].wait_recv()
        for k in range(nx):
            xsends[k].wait_send()
        for k in range(NF):
            fwds[k].wait_send()
        local_copy.wait()

    x = pltpu.with_memory_space_constraint(x, pltpu.MemorySpace.HBM)
    return pl.pallas_call(
        body,
        out_shape=jax.ShapeDtypeStruct((2 * m, n_half), x.dtype),
        in_specs=[pl.BlockSpec(memory_space=pltpu.MemorySpace.HBM)],
        out_specs=pl.BlockSpec(memory_space=pltpu.MemorySpace.VMEM),
        scratch_shapes=[
            pltpu.VMEM((2, m // NCHUNK, n // 2), x.dtype),
            pltpu.SemaphoreType.DMA((NF + ND,)),
            pltpu.SemaphoreType.DMA((NF + ND,)),
            pltpu.SemaphoreType.DMA((NF,)),
            pltpu.SemaphoreType.DMA((NF,)),
            pltpu.SemaphoreType.DMA,
            pltpu.SemaphoreType.DMA((2,)),
        ],
        compiler_params=pltpu.CompilerParams(collective_id=0),
    )(x)
